# Initial kernel scaffold; baseline (speedup 1.0000x reference)
import jax
import jax.numpy as jnp
from jax.experimental import pallas as pl


def kernel(inputs, edge_index, W, b):
    raise NotImplementedError("write your pallas kernel here")


# speedup vs baseline: 7.5171x; 7.5171x over previous
import functools

import jax
import jax.numpy as jnp
from jax import lax
from jax.experimental import pallas as pl
from jax.experimental.pallas import tpu as pltpu
from jax.experimental.pallas import tpu_sc as plsc

N_NODES = 10000
N_EDGES = 320000
D = 128

NC = 2
NS = 16
NW = NC * NS

E_PER_W = N_EDGES // NW
CHUNK = 80
NCHUNK = E_PER_W // CHUNK
ROWS_A = 624
TAIL = N_NODES - NS * ROWS_A
ZROWS = 48


def _sc_segment_sum(x, src_r, dst_r):
  mesh = plsc.VectorSubcoreMesh(core_axis_name="c", subcore_axis_name="s")

  @functools.partial(
      pl.kernel,
      mesh=mesh,
      out_type=jax.ShapeDtypeStruct((NC, N_NODES, D), jnp.float32),
      scratch_types=dict(
          src_v=pltpu.VMEM((NCHUNK, CHUNK), jnp.int32),
          dst_v=pltpu.VMEM((NCHUNK, CHUNK), jnp.int32),
          rows_v=pltpu.VMEM((CHUNK, D), jnp.float32),
          zbuf=pltpu.VMEM((ZROWS, D), jnp.float32),
          acc=pltpu.VMEM_SHARED((N_NODES, D), jnp.float32),
          sem=pltpu.SemaphoreType.DMA,
      ),
  )
  def seg_sum(x_hbm, src_hbm, dst_hbm, out_hbm, *, src_v, dst_v, rows_v,
              zbuf, acc, sem):
    c = lax.axis_index("c")
    s = lax.axis_index("s")
    wid = s * NC + c

    def zrow(i, _):
      for k in range(D // 16):
        zbuf[i, pl.ds(k * 16, 16)] = jnp.zeros((16,), jnp.float32)
      return 0
    lax.fori_loop(0, ZROWS, zrow, 0)
    base = s * ROWS_A
    for t in range(ROWS_A // ZROWS):
      pltpu.sync_copy(zbuf, acc.at[pl.ds(base + t * ZROWS, ZROWS)])

    @pl.when(s == NS - 1)
    def _zero_tail():
      pltpu.sync_copy(zbuf.at[pl.ds(0, TAIL)],
                      acc.at[pl.ds(NS * ROWS_A, TAIL)])

    plsc.subcore_barrier()

    pltpu.sync_copy(src_hbm.at[wid], src_v)
    pltpu.sync_copy(dst_hbm.at[wid], dst_v)

    def body(j, _):
      pltpu.async_copy(x_hbm.at[src_v.at[j]], rows_v, sem).wait()
      pltpu.sync_copy(rows_v, acc.at[dst_v.at[j]], add=True)
      return 0
    lax.fori_loop(0, NCHUNK, body, 0)

    plsc.subcore_barrier()
    pltpu.sync_copy(acc.at[pl.ds(base, ROWS_A)],
                    out_hbm.at[c, pl.ds(base, ROWS_A)])

    @pl.when(s == NS - 1)
    def _copy_tail():
      pltpu.sync_copy(acc.at[pl.ds(NS * ROWS_A, TAIL)],
                      out_hbm.at[c, pl.ds(NS * ROWS_A, TAIL)])

  return seg_sum(x, src_r, dst_r)


def _tc_linear(p0, p1, W, b2):
  blk = 1000

  def body(p0_ref, p1_ref, w_ref, b_ref, o_ref):
    h = p0_ref[...] + p1_ref[...]
    o_ref[...] = lax.dot_general(
        h, w_ref[...], (((1,), (1,)), ((), ())),
        preferred_element_type=jnp.float32) + b_ref[...]

  return pl.pallas_call(
      body,
      grid=(N_NODES // blk,),
      in_specs=[
          pl.BlockSpec((blk, D), lambda i: (i, 0)),
          pl.BlockSpec((blk, D), lambda i: (i, 0)),
          pl.BlockSpec((D, D), lambda i: (0, 0)),
          pl.BlockSpec((1, D), lambda i: (0, 0)),
      ],
      out_specs=pl.BlockSpec((blk, D), lambda i: (i, 0)),
      out_shape=jax.ShapeDtypeStruct((N_NODES, D), jnp.float32),
  )(p0, p1, W, b2)


@jax.jit
def kernel(inputs, edge_index, W, b):
  src_r = edge_index[0].reshape(NW, NCHUNK, CHUNK)
  dst_r = edge_index[1].reshape(NW, NCHUNK, CHUNK)
  partials = _sc_segment_sum(inputs, src_r, dst_r)
  return _tc_linear(partials[0], partials[1], W, b.reshape(1, D))


# speedup vs baseline: 11.4206x; 1.5193x over previous
import functools

import jax
import jax.numpy as jnp
from jax import lax
from jax.experimental import pallas as pl
from jax.experimental.pallas import tpu as pltpu
from jax.experimental.pallas import tpu_sc as plsc

N_NODES = 10000
N_EDGES = 320000
D = 128

NC = 2
NS = 16
NW = NC * NS

E_PER_W = N_EDGES // NW
CHUNK = 80
NCHUNK = E_PER_W // CHUNK
ROWS_A = 624
TAIL = N_NODES - NS * ROWS_A
ZROWS = 16


def _sc_segment_sum(x, src_r, dst_r):
  mesh = plsc.VectorSubcoreMesh(core_axis_name="c", subcore_axis_name="s")

  @functools.partial(
      pl.kernel,
      mesh=mesh,
      out_type=jax.ShapeDtypeStruct((NC, N_NODES, D), jnp.float32),
      scratch_types=dict(
          src_v=pltpu.VMEM((E_PER_W,), jnp.int32),
          dst_v=pltpu.VMEM((NCHUNK, CHUNK), jnp.int32),
          rows0=pltpu.VMEM((CHUNK, D), jnp.float32),
          rows1=pltpu.VMEM((CHUNK, D), jnp.float32),
          zbuf=pltpu.VMEM((ZROWS, D), jnp.float32),
          acc=pltpu.VMEM_SHARED((N_NODES, D), jnp.float32),
          sem0=pltpu.SemaphoreType.DMA,
          sem1=pltpu.SemaphoreType.DMA,
      ),
  )
  def seg_sum(x_hbm, src_hbm, dst_hbm, out_hbm, *, src_v, dst_v, rows0,
              rows1, zbuf, acc, sem0, sem1):
    c = lax.axis_index("c")
    s = lax.axis_index("s")
    wid = s * NC + c

    def zrow(i, _):
      for k in range(D // 16):
        zbuf[i, pl.ds(k * 16, 16)] = jnp.zeros((16,), jnp.float32)
      return 0
    lax.fori_loop(0, ZROWS, zrow, 0)
    base = s * ROWS_A
    for t in range(ROWS_A // ZROWS):
      pltpu.sync_copy(zbuf, acc.at[pl.ds(base + t * ZROWS, ZROWS)])

    @pl.when(s == NS - 1)
    def _zero_tail():
      pltpu.sync_copy(zbuf.at[pl.ds(0, TAIL)],
                      acc.at[pl.ds(NS * ROWS_A, TAIL)])

    plsc.subcore_barrier()

    pltpu.sync_copy(src_hbm.at[wid], src_v)
    pltpu.sync_copy(dst_hbm.at[wid], dst_v)

    def sidx(j):
      return src_v.at[pl.ds(pl.multiple_of(j * CHUNK, CHUNK), CHUNK)]

    pltpu.async_copy(x_hbm.at[sidx(0)], rows0, sem0)

    def body(i, _):
      j = 2 * i
      pltpu.async_copy(x_hbm.at[sidx(j + 1)], rows1, sem1)
      pltpu.make_async_copy(x_hbm.at[sidx(j)], rows0, sem0).wait()
      pltpu.sync_copy(rows0, acc.at[dst_v.at[j]], add=True)
      pltpu.async_copy(x_hbm.at[sidx(j + 2)], rows0, sem0)
      pltpu.make_async_copy(x_hbm.at[sidx(j + 1)], rows1, sem1).wait()
      pltpu.sync_copy(rows1, acc.at[dst_v.at[j + 1]], add=True)
      return 0
    lax.fori_loop(0, (NCHUNK - 1) // 2, body, 0)

    pltpu.make_async_copy(x_hbm.at[sidx(NCHUNK - 1)], rows0, sem0).wait()
    pltpu.sync_copy(rows0, acc.at[dst_v.at[NCHUNK - 1]], add=True)

    plsc.subcore_barrier()
    pltpu.sync_copy(acc.at[pl.ds(base, ROWS_A)],
                    out_hbm.at[c, pl.ds(base, ROWS_A)])

    @pl.when(s == NS - 1)
    def _copy_tail():
      pltpu.sync_copy(acc.at[pl.ds(NS * ROWS_A, TAIL)],
                      out_hbm.at[c, pl.ds(NS * ROWS_A, TAIL)])

  return seg_sum(x, src_r, dst_r)


def _tc_linear(p0, p1, W, b2):
  blk = 1000

  def body(p0_ref, p1_ref, w_ref, b_ref, o_ref):
    h = p0_ref[...] + p1_ref[...]
    o_ref[...] = lax.dot_general(
        h, w_ref[...], (((1,), (1,)), ((), ())),
        preferred_element_type=jnp.float32) + b_ref[...]

  return pl.pallas_call(
      body,
      grid=(N_NODES // blk,),
      in_specs=[
          pl.BlockSpec((blk, D), lambda i: (i, 0)),
          pl.BlockSpec((blk, D), lambda i: (i, 0)),
          pl.BlockSpec((D, D), lambda i: (0, 0)),
          pl.BlockSpec((1, D), lambda i: (0, 0)),
      ],
      out_specs=pl.BlockSpec((blk, D), lambda i: (i, 0)),
      out_shape=jax.ShapeDtypeStruct((N_NODES, D), jnp.float32),
  )(p0, p1, W, b2)


@jax.jit
def kernel(inputs, edge_index, W, b):
  src_r = edge_index[0].reshape(NW, E_PER_W)
  dst_r = edge_index[1].reshape(NW, NCHUNK, CHUNK)
  partials = _sc_segment_sum(inputs, src_r, dst_r)
  return _tc_linear(partials[0], partials[1], W, b.reshape(1, D))


# speedup vs baseline: 11.7379x; 1.0278x over previous
import functools

import jax
import jax.numpy as jnp
from jax import lax
from jax.experimental import pallas as pl
from jax.experimental.pallas import tpu as pltpu
from jax.experimental.pallas import tpu_sc as plsc

N_NODES = 10000
N_EDGES = 320000
D = 128

NC = 2
NS = 16
NW = NC * NS

E_PER_W = N_EDGES // NW
CHUNK = 80
NCHUNK = E_PER_W // CHUNK
ROWS_A = 624
TAIL = N_NODES - NS * ROWS_A
ZROWS = 16


def _sc_segment_sum(x, src_r, dst_r):
  mesh = plsc.VectorSubcoreMesh(core_axis_name="c", subcore_axis_name="s")

  @functools.partial(
      pl.kernel,
      mesh=mesh,
      out_type=jax.ShapeDtypeStruct((NC, N_NODES, D), jnp.float32),
      scratch_types=dict(
          src_v=pltpu.VMEM((E_PER_W,), jnp.int32),
          dst_v=pltpu.VMEM((NCHUNK, CHUNK), jnp.int32),
          rows0=pltpu.VMEM((CHUNK, D), jnp.float32),
          rows1=pltpu.VMEM((CHUNK, D), jnp.float32),
          zbuf=pltpu.VMEM((ZROWS, D), jnp.float32),
          acc=pltpu.VMEM_SHARED((N_NODES, D), jnp.float32),
          sem0=pltpu.SemaphoreType.DMA,
          sem1=pltpu.SemaphoreType.DMA,
          semz=pltpu.SemaphoreType.DMA,
      ),
  )
  def seg_sum(x_hbm, src_hbm, dst_hbm, out_hbm, *, src_v, dst_v, rows0,
              rows1, zbuf, acc, sem0, sem1, semz):
    c = lax.axis_index("c")
    s = lax.axis_index("s")
    wid = s * NC + c

    idx_src = pltpu.async_copy(src_hbm.at[wid], src_v, sem0)
    idx_dst = pltpu.async_copy(dst_hbm.at[wid], dst_v, sem1)

    def zrow(i, _):
      for k in range(D // 16):
        zbuf[i, pl.ds(k * 16, 16)] = jnp.zeros((16,), jnp.float32)
      return 0
    lax.fori_loop(0, ZROWS, zrow, 0)
    base = s * ROWS_A
    zcopies = [
        pltpu.async_copy(zbuf, acc.at[pl.ds(base + t * ZROWS, ZROWS)], semz)
        for t in range(ROWS_A // ZROWS)
    ]

    @pl.when(s == NS - 1)
    def _zero_tail():
      pltpu.async_copy(zbuf.at[pl.ds(0, TAIL)],
                       acc.at[pl.ds(NS * ROWS_A, TAIL)], semz).wait()

    for zc in zcopies:
      zc.wait()
    plsc.subcore_barrier()
    idx_src.wait()
    idx_dst.wait()

    def sidx(j):
      return src_v.at[pl.ds(pl.multiple_of(j * CHUNK, CHUNK), CHUNK)]

    pltpu.async_copy(x_hbm.at[sidx(0)], rows0, sem0)

    def body(i, _):
      j = 2 * i
      pltpu.async_copy(x_hbm.at[sidx(j + 1)], rows1, sem1)
      pltpu.make_async_copy(x_hbm.at[sidx(j)], rows0, sem0).wait()
      pltpu.sync_copy(rows0, acc.at[dst_v.at[j]], add=True)
      pltpu.async_copy(x_hbm.at[sidx(j + 2)], rows0, sem0)
      pltpu.make_async_copy(x_hbm.at[sidx(j + 1)], rows1, sem1).wait()
      pltpu.sync_copy(rows1, acc.at[dst_v.at[j + 1]], add=True)
      return 0
    lax.fori_loop(0, (NCHUNK - 1) // 2, body, 0)

    pltpu.make_async_copy(x_hbm.at[sidx(NCHUNK - 1)], rows0, sem0).wait()
    pltpu.sync_copy(rows0, acc.at[dst_v.at[NCHUNK - 1]], add=True)

    plsc.subcore_barrier()
    pltpu.sync_copy(acc.at[pl.ds(base, ROWS_A)],
                    out_hbm.at[c, pl.ds(base, ROWS_A)])

    @pl.when(s == NS - 1)
    def _copy_tail():
      pltpu.sync_copy(acc.at[pl.ds(NS * ROWS_A, TAIL)],
                      out_hbm.at[c, pl.ds(NS * ROWS_A, TAIL)])

  return seg_sum(x, src_r, dst_r)


def _tc_linear(p0, p1, W, b2):
  blk = 1000

  def body(p0_ref, p1_ref, w_ref, b_ref, o_ref):
    h = p0_ref[...] + p1_ref[...]
    o_ref[...] = lax.dot_general(
        h, w_ref[...], (((1,), (1,)), ((), ())),
        preferred_element_type=jnp.float32) + b_ref[...]

  return pl.pallas_call(
      body,
      grid=(N_NODES // blk,),
      in_specs=[
          pl.BlockSpec((blk, D), lambda i: (i, 0)),
          pl.BlockSpec((blk, D), lambda i: (i, 0)),
          pl.BlockSpec((D, D), lambda i: (0, 0)),
          pl.BlockSpec((1, D), lambda i: (0, 0)),
      ],
      out_specs=pl.BlockSpec((blk, D), lambda i: (i, 0)),
      out_shape=jax.ShapeDtypeStruct((N_NODES, D), jnp.float32),
  )(p0, p1, W, b2)


@jax.jit
def kernel(inputs, edge_index, W, b):
  src_r = edge_index[0].reshape(NW, E_PER_W)
  dst_r = edge_index[1].reshape(NW, NCHUNK, CHUNK)
  partials = _sc_segment_sum(inputs, src_r, dst_r)
  return _tc_linear(partials[0], partials[1], W, b.reshape(1, D))


# speedup vs baseline: 12.7089x; 1.0827x over previous
import functools

import jax
import jax.numpy as jnp
from jax import lax
from jax.experimental import pallas as pl
from jax.experimental.pallas import tpu as pltpu
from jax.experimental.pallas import tpu_sc as plsc

N_NODES = 10000
N_EDGES = 320000
D = 128

NC = 2
NS = 16
NW = NC * NS

E_PER_W = N_EDGES // NW
CHUNK = 80
NCHUNK = E_PER_W // CHUNK
NSLOT = 4
ROWS_A = 624
TAIL = N_NODES - NS * ROWS_A
ZROWS = 16


def _sc_segment_sum(x, src_r, dst_r):
  mesh = plsc.VectorSubcoreMesh(core_axis_name="c", subcore_axis_name="s")

  @functools.partial(
      pl.kernel,
      mesh=mesh,
      out_type=jax.ShapeDtypeStruct((NC, N_NODES, D), jnp.float32),
      scratch_types=dict(
          sring=pltpu.VMEM((NSLOT * CHUNK,), jnp.int32),
          dring=pltpu.VMEM((NSLOT, CHUNK), jnp.int32),
          rows0=pltpu.VMEM((CHUNK, D), jnp.float32),
          rows1=pltpu.VMEM((CHUNK, D), jnp.float32),
          rows2=pltpu.VMEM((CHUNK, D), jnp.float32),
          rows3=pltpu.VMEM((CHUNK, D), jnp.float32),
          zbuf=pltpu.VMEM((ZROWS, D), jnp.float32),
          acc=pltpu.VMEM_SHARED((N_NODES, D), jnp.float32),
          g0=pltpu.SemaphoreType.DMA,
          g1=pltpu.SemaphoreType.DMA,
          g2=pltpu.SemaphoreType.DMA,
          g3=pltpu.SemaphoreType.DMA,
          s0=pltpu.SemaphoreType.DMA,
          s1=pltpu.SemaphoreType.DMA,
          s2=pltpu.SemaphoreType.DMA,
          s3=pltpu.SemaphoreType.DMA,
          si=pltpu.SemaphoreType.DMA,
          sd=pltpu.SemaphoreType.DMA,
          semz=pltpu.SemaphoreType.DMA,
      ),
  )
  def seg_sum(x_hbm, src_hbm, dst_hbm, out_hbm, *, sring, dring, rows0,
              rows1, rows2, rows3, zbuf, acc, g0, g1, g2, g3, s0, s1, s2,
              s3, si, sd, semz):
    c = lax.axis_index("c")
    s_ax = lax.axis_index("s")
    wid = s_ax * NC + c
    rows = [rows0, rows1, rows2, rows3]
    semg = [g0, g1, g2, g3]
    sems = [s0, s1, s2, s3]

    def zrow(i, _):
      for k in range(D // 16):
        zbuf[i, pl.ds(k * 16, 16)] = jnp.zeros((16,), jnp.float32)
      return 0
    lax.fori_loop(0, ZROWS, zrow, 0)
    base = s_ax * ROWS_A
    zcopies = [
        pltpu.async_copy(zbuf, acc.at[pl.ds(base + t * ZROWS, ZROWS)], semz)
        for t in range(ROWS_A // ZROWS)
    ]

    @pl.when(s_ax == NS - 1)
    def _zero_tail():
      pltpu.async_copy(zbuf.at[pl.ds(0, TAIL)],
                       acc.at[pl.ds(NS * ROWS_A, TAIL)], semz).wait()

    for zc in zcopies:
      zc.wait()
    plsc.subcore_barrier()

    def sl(k):
      return sring.at[pl.ds(k * CHUNK, CHUNK)]

    def load_pair(jj, m):
      off = pl.multiple_of(wid * E_PER_W + jj * CHUNK, CHUNK)
      pltpu.async_copy(src_hbm.at[pl.ds(off, CHUNK)], sl(m), si)
      pltpu.async_copy(dst_hbm.at[pl.ds(off, CHUNK)], dring.at[m], sd)

    def drain_si():
      pltpu.make_async_copy(src_hbm.at[pl.ds(0, CHUNK)], sl(0), si).wait()

    def drain_sd():
      pltpu.make_async_copy(dst_hbm.at[pl.ds(0, CHUNK)], dring.at[0],
                            sd).wait()

    def g_issue(k):
      pltpu.async_copy(x_hbm.at[sl(k)], rows[k], semg[k])

    def g_wait(k):
      pltpu.make_async_copy(x_hbm.at[sl(k)], rows[k], semg[k]).wait()

    def s_issue(k):
      pltpu.async_copy(rows[k], acc.at[dring.at[k]], sems[k], add=True)

    def s_wait(k):
      pltpu.make_async_copy(rows[k], acc.at[dring.at[k]], sems[k]).wait()

    for t in range(3):
      load_pair(t, t)
    drain_si()
    g_issue(0)
    drain_si()
    g_issue(1)

    g_wait(0)
    drain_sd()
    s_issue(0)
    load_pair(3, 3)
    drain_si()
    g_issue(2)

    def body(i, _):
      jb = 4 * i + 1
      for kk in range(4):
        j = jb + kk
        k = (1 + kk) % 4
        g_wait(k)
        drain_sd()
        s_issue(k)
        s_wait((k - 1) % 4)
        load_pair(j + 3, (k - 1) % 4)
        drain_si()
        g_issue((k + 2) % 4)
      return 0
    lax.fori_loop(0, 30, body, 0)

    g_wait(1)
    drain_sd()
    s_issue(1)
    s_wait(0)
    load_pair(124, 0)
    drain_si()
    g_issue(3)

    g_wait(2)
    drain_sd()
    s_issue(2)
    s_wait(1)
    drain_si()
    g_issue(0)

    g_wait(3)
    drain_sd()
    s_issue(3)
    s_wait(2)

    g_wait(0)
    drain_sd()
    s_issue(0)
    s_wait(3)
    s_wait(0)

    plsc.subcore_barrier()
    pltpu.sync_copy(acc.at[pl.ds(base, ROWS_A)],
                    out_hbm.at[c, pl.ds(base, ROWS_A)])

    @pl.when(s_ax == NS - 1)
    def _copy_tail():
      pltpu.sync_copy(acc.at[pl.ds(NS * ROWS_A, TAIL)],
                      out_hbm.at[c, pl.ds(NS * ROWS_A, TAIL)])

  return seg_sum(x, src_r, dst_r)


def _tc_linear(p0, p1, W, b2):
  blk = 1000

  def body(p0_ref, p1_ref, w_ref, b_ref, o_ref):
    h = p0_ref[...] + p1_ref[...]
    o_ref[...] = lax.dot_general(
        h, w_ref[...], (((1,), (1,)), ((), ())),
        preferred_element_type=jnp.float32) + b_ref[...]

  return pl.pallas_call(
      body,
      grid=(N_NODES // blk,),
      in_specs=[
          pl.BlockSpec((blk, D), lambda i: (i, 0)),
          pl.BlockSpec((blk, D), lambda i: (i, 0)),
          pl.BlockSpec((D, D), lambda i: (0, 0)),
          pl.BlockSpec((1, D), lambda i: (0, 0)),
      ],
      out_specs=pl.BlockSpec((blk, D), lambda i: (i, 0)),
      out_shape=jax.ShapeDtypeStruct((N_NODES, D), jnp.float32),
  )(p0, p1, W, b2)


@jax.jit
def kernel(inputs, edge_index, W, b):
  src_r = edge_index[0]
  dst_r = edge_index[1]
  partials = _sc_segment_sum(inputs, src_r, dst_r)
  return _tc_linear(partials[0], partials[1], W, b.reshape(1, D))


# speedup vs baseline: 12.7946x; 1.0067x over previous
import functools

import jax
import jax.numpy as jnp
from jax import lax
from jax.experimental import pallas as pl
from jax.experimental.pallas import tpu as pltpu
from jax.experimental.pallas import tpu_sc as plsc

N_NODES = 10000
N_EDGES = 320000
D = 128

NC = 2
NS = 16
NW = NC * NS

E_PER_W = N_EDGES // NW
CHUNK = 80
NCHUNK = E_PER_W // CHUNK
NSLOT = 4
ROWS_A = 624
TAIL = N_NODES - NS * ROWS_A
ZROWS = 16


def _sc_segment_sum(x, src_r, dst_r):
  mesh = plsc.VectorSubcoreMesh(core_axis_name="c", subcore_axis_name="s")

  @functools.partial(
      pl.kernel,
      mesh=mesh,
      out_type=jax.ShapeDtypeStruct((NC, N_NODES, D), jnp.float32),
      scratch_types=dict(
          sring=pltpu.VMEM((NSLOT * CHUNK,), jnp.int32),
          dring=pltpu.VMEM((NSLOT, CHUNK), jnp.int32),
          rows0=pltpu.VMEM((CHUNK, D), jnp.float32),
          rows1=pltpu.VMEM((CHUNK, D), jnp.float32),
          rows2=pltpu.VMEM((CHUNK, D), jnp.float32),
          rows3=pltpu.VMEM((CHUNK, D), jnp.float32),
          zbuf=pltpu.VMEM((ZROWS, D), jnp.float32),
          acc=pltpu.VMEM_SHARED((N_NODES, D), jnp.float32),
          g0=pltpu.SemaphoreType.DMA,
          g1=pltpu.SemaphoreType.DMA,
          g2=pltpu.SemaphoreType.DMA,
          g3=pltpu.SemaphoreType.DMA,
          s0=pltpu.SemaphoreType.DMA,
          s1=pltpu.SemaphoreType.DMA,
          s2=pltpu.SemaphoreType.DMA,
          s3=pltpu.SemaphoreType.DMA,
          si=pltpu.SemaphoreType.DMA,
          sd=pltpu.SemaphoreType.DMA,
          semz=pltpu.SemaphoreType.DMA,
      ),
  )
  def seg_sum(x_hbm, src_hbm, dst_hbm, out_hbm, *, sring, dring, rows0,
              rows1, rows2, rows3, zbuf, acc, g0, g1, g2, g3, s0, s1, s2,
              s3, si, sd, semz):
    c = lax.axis_index("c")
    s_ax = lax.axis_index("s")
    wid = s_ax * NC + c
    rows = [rows0, rows1, rows2, rows3]
    semg = [g0, g1, g2, g3]
    sems = [s0, s1, s2, s3]

    def sl(k):
      return sring.at[pl.ds(k * CHUNK, CHUNK)]

    def load_pair(jj, m):
      off = pl.multiple_of(wid * E_PER_W + jj * CHUNK, CHUNK)
      pltpu.async_copy(src_hbm.at[pl.ds(off, CHUNK)], sl(m), si)
      pltpu.async_copy(dst_hbm.at[pl.ds(off, CHUNK)], dring.at[m], sd)

    def drain_si():
      pltpu.make_async_copy(src_hbm.at[pl.ds(0, CHUNK)], sl(0), si).wait()

    def drain_sd():
      pltpu.make_async_copy(dst_hbm.at[pl.ds(0, CHUNK)], dring.at[0],
                            sd).wait()

    def g_issue(k):
      pltpu.async_copy(x_hbm.at[sl(k)], rows[k], semg[k])

    def g_wait(k):
      pltpu.make_async_copy(x_hbm.at[sl(k)], rows[k], semg[k]).wait()

    def s_issue(k):
      pltpu.async_copy(rows[k], acc.at[dring.at[k]], sems[k], add=True)

    def s_wait(k):
      pltpu.make_async_copy(rows[k], acc.at[dring.at[k]], sems[k]).wait()

    for t in range(3):
      load_pair(t, t)
    drain_si()
    g_issue(0)
    drain_si()
    g_issue(1)

    for i in range(ZROWS):
      for k in range(D // 16):
        zbuf[i, pl.ds(k * 16, 16)] = jnp.zeros((16,), jnp.float32)
    base = s_ax * ROWS_A
    zcopies = [
        pltpu.async_copy(zbuf, acc.at[pl.ds(base + t * ZROWS, ZROWS)], semz)
        for t in range(ROWS_A // ZROWS)
    ]

    @pl.when(s_ax == NS - 1)
    def _zero_tail():
      pltpu.async_copy(zbuf.at[pl.ds(0, TAIL)],
                       acc.at[pl.ds(NS * ROWS_A, TAIL)], semz).wait()

    for zc in zcopies:
      zc.wait()
    plsc.subcore_barrier()

    g_wait(0)
    drain_sd()
    s_issue(0)
    load_pair(3, 3)
    drain_si()
    g_issue(2)

    def body(i, _):
      jb = 4 * i + 1
      for kk in range(4):
        j = jb + kk
        k = (1 + kk) % 4
        g_wait(k)
        drain_sd()
        s_issue(k)
        s_wait((k - 1) % 4)
        load_pair(j + 3, (k - 1) % 4)
        drain_si()
        g_issue((k + 2) % 4)
      return 0
    lax.fori_loop(0, 30, body, 0)

    g_wait(1)
    drain_sd()
    s_issue(1)
    s_wait(0)
    load_pair(124, 0)
    drain_si()
    g_issue(3)

    g_wait(2)
    drain_sd()
    s_issue(2)
    s_wait(1)
    drain_si()
    g_issue(0)

    g_wait(3)
    drain_sd()
    s_issue(3)
    s_wait(2)

    g_wait(0)
    drain_sd()
    s_issue(0)
    s_wait(3)
    s_wait(0)

    plsc.subcore_barrier()
    pltpu.sync_copy(acc.at[pl.ds(base, ROWS_A)],
                    out_hbm.at[c, pl.ds(base, ROWS_A)])

    @pl.when(s_ax == NS - 1)
    def _copy_tail():
      pltpu.sync_copy(acc.at[pl.ds(NS * ROWS_A, TAIL)],
                      out_hbm.at[c, pl.ds(NS * ROWS_A, TAIL)])

  return seg_sum(x, src_r, dst_r)


def _tc_linear(p0, p1, W, b2):
  blk = 1000

  def body(p0_ref, p1_ref, w_ref, b_ref, o_ref):
    h = p0_ref[...] + p1_ref[...]
    o_ref[...] = lax.dot_general(
        h, w_ref[...], (((1,), (1,)), ((), ())),
        preferred_element_type=jnp.float32) + b_ref[...]

  return pl.pallas_call(
      body,
      grid=(N_NODES // blk,),
      in_specs=[
          pl.BlockSpec((blk, D), lambda i: (i, 0)),
          pl.BlockSpec((blk, D), lambda i: (i, 0)),
          pl.BlockSpec((D, D), lambda i: (0, 0)),
          pl.BlockSpec((1, D), lambda i: (0, 0)),
      ],
      out_specs=pl.BlockSpec((blk, D), lambda i: (i, 0)),
      out_shape=jax.ShapeDtypeStruct((N_NODES, D), jnp.float32),
  )(p0, p1, W, b2)


@jax.jit
def kernel(inputs, edge_index, W, b):
  src_r = edge_index[0]
  dst_r = edge_index[1]
  partials = _sc_segment_sum(inputs, src_r, dst_r)
  return _tc_linear(partials[0], partials[1], W, b.reshape(1, D))


# speedup vs baseline: 13.4577x; 1.0518x over previous
import functools

import jax
import jax.numpy as jnp
from jax import lax
from jax.experimental import pallas as pl
from jax.experimental.pallas import tpu as pltpu
from jax.experimental.pallas import tpu_sc as plsc

N_NODES = 10000
N_EDGES = 320000
D = 128

NC = 2
NS = 16
NW = NC * NS

E_PER_W = N_EDGES // NW
CHUNK = 80
NCHUNK = E_PER_W // CHUNK
NSLOT = 4
ROWS_A = 624
TAIL = N_NODES - NS * ROWS_A
ZROWS = 16


def _sc_segment_sum(x, src_r, dst_r):
  mesh = plsc.VectorSubcoreMesh(core_axis_name="c", subcore_axis_name="s")

  @functools.partial(
      pl.kernel,
      mesh=mesh,
      out_type=jax.ShapeDtypeStruct((NC, N_NODES, D), jnp.float32),
      scratch_types=dict(
          sring=pltpu.VMEM((NSLOT * CHUNK,), jnp.int32),
          dring=pltpu.VMEM((NSLOT, CHUNK), jnp.int32),
          rows0=pltpu.VMEM((CHUNK, D), jnp.float32),
          rows1=pltpu.VMEM((CHUNK, D), jnp.float32),
          rows2=pltpu.VMEM((CHUNK, D), jnp.float32),
          rows3=pltpu.VMEM((CHUNK, D), jnp.float32),
          zbuf=pltpu.VMEM((ZROWS, D), jnp.float32),
          acc=pltpu.VMEM_SHARED((N_NODES, D), jnp.float32),
          g0=pltpu.SemaphoreType.DMA,
          g1=pltpu.SemaphoreType.DMA,
          g2=pltpu.SemaphoreType.DMA,
          g3=pltpu.SemaphoreType.DMA,
          s0=pltpu.SemaphoreType.DMA,
          s1=pltpu.SemaphoreType.DMA,
          s2=pltpu.SemaphoreType.DMA,
          s3=pltpu.SemaphoreType.DMA,
          si=pltpu.SemaphoreType.DMA,
          sd=pltpu.SemaphoreType.DMA,
          semz=pltpu.SemaphoreType.DMA,
      ),
  )
  def seg_sum(x_hbm, src_hbm, dst_hbm, out_hbm, *, sring, dring, rows0,
              rows1, rows2, rows3, zbuf, acc, g0, g1, g2, g3, s0, s1, s2,
              s3, si, sd, semz):
    c = lax.axis_index("c")
    s_ax = lax.axis_index("s")
    wid = s_ax * NC + c
    rows = [rows0, rows1, rows2, rows3]
    semg = [g0, g1, g2, g3]
    sems = [s0, s1, s2, s3]

    def sl(k):
      return sring.at[pl.ds(k * CHUNK, CHUNK)]

    def load_pair(jj, m):
      off = pl.multiple_of(wid * E_PER_W + jj * CHUNK, CHUNK)
      pltpu.async_copy(src_hbm.at[pl.ds(off, CHUNK)], sl(m), si)
      pltpu.async_copy(dst_hbm.at[pl.ds(off, CHUNK)], dring.at[m], sd)

    def drain_si():
      pltpu.make_async_copy(src_hbm.at[pl.ds(0, CHUNK)], sl(0), si).wait()

    def drain_sd():
      pltpu.make_async_copy(dst_hbm.at[pl.ds(0, CHUNK)], dring.at[0],
                            sd).wait()

    def g_issue(k):
      pltpu.async_copy(x_hbm.at[sl(k)], rows[k], semg[k])

    def g_wait(k):
      pltpu.make_async_copy(x_hbm.at[sl(k)], rows[k], semg[k]).wait()

    def s_issue(k):
      pltpu.async_copy(rows[k], acc.at[dring.at[k]], sems[k], add=True)

    def s_wait(k):
      pltpu.make_async_copy(rows[k], acc.at[dring.at[k]], sems[k]).wait()

    for t in range(3):
      load_pair(t, t)
    drain_si()
    g_issue(0)
    drain_si()
    g_issue(1)

    for i in range(ZROWS):
      for k in range(D // 16):
        zbuf[i, pl.ds(k * 16, 16)] = jnp.zeros((16,), jnp.float32)
    base = s_ax * ROWS_A
    zcopies = [
        pltpu.async_copy(zbuf, acc.at[pl.ds(base + t * ZROWS, ZROWS)], semz)
        for t in range(ROWS_A // ZROWS)
    ]

    @pl.when(s_ax == NS - 1)
    def _zero_tail():
      pltpu.async_copy(zbuf.at[pl.ds(0, TAIL)],
                       acc.at[pl.ds(NS * ROWS_A, TAIL)], semz).wait()

    for zc in zcopies:
      zc.wait()
    plsc.subcore_barrier()

    g_wait(0)
    drain_sd()
    s_issue(0)
    load_pair(3, 3)
    drain_si()
    g_issue(2)

    def body(i, _):
      jb = 4 * i + 1
      for kk in range(4):
        j = jb + kk
        k = (1 + kk) % 4
        g_wait(k)
        drain_sd()
        s_issue(k)
        s_wait((k - 1) % 4)
        load_pair(j + 3, (k - 1) % 4)
        drain_si()
        g_issue((k + 2) % 4)
      return 0
    lax.fori_loop(0, 30, body, 0)

    g_wait(1)
    drain_sd()
    s_issue(1)
    s_wait(0)
    load_pair(124, 0)
    drain_si()
    g_issue(3)

    g_wait(2)
    drain_sd()
    s_issue(2)
    s_wait(1)
    drain_si()
    g_issue(0)

    g_wait(3)
    drain_sd()
    s_issue(3)
    s_wait(2)

    g_wait(0)
    drain_sd()
    s_issue(0)
    s_wait(3)
    s_wait(0)

    plsc.subcore_barrier()
    pltpu.sync_copy(acc.at[pl.ds(base, ROWS_A)],
                    out_hbm.at[c, pl.ds(base, ROWS_A)])

    @pl.when(s_ax == NS - 1)
    def _copy_tail():
      pltpu.sync_copy(acc.at[pl.ds(NS * ROWS_A, TAIL)],
                      out_hbm.at[c, pl.ds(NS * ROWS_A, TAIL)])

  return seg_sum(x, src_r, dst_r)


def _tc_linear(p, W, b2):
  blk = 1000

  def body(p_ref, w_ref, b_ref, o_ref):
    h = p_ref[0] + p_ref[1]
    o_ref[...] = lax.dot_general(
        h, w_ref[...], (((1,), (1,)), ((), ())),
        preferred_element_type=jnp.float32) + b_ref[...]

  return pl.pallas_call(
      body,
      grid=(N_NODES // blk,),
      in_specs=[
          pl.BlockSpec((NC, blk, D), lambda i: (0, i, 0)),
          pl.BlockSpec((D, D), lambda i: (0, 0)),
          pl.BlockSpec((1, D), lambda i: (0, 0)),
      ],
      out_specs=pl.BlockSpec((blk, D), lambda i: (i, 0)),
      out_shape=jax.ShapeDtypeStruct((N_NODES, D), jnp.float32),
  )(p, W, b2)


@jax.jit
def kernel(inputs, edge_index, W, b):
  src_r = edge_index[0]
  dst_r = edge_index[1]
  partials = _sc_segment_sum(inputs, src_r, dst_r)
  return _tc_linear(partials, W, b.reshape(1, D))


# speedup vs baseline: 14.4388x; 1.0729x over previous
import functools

import jax
import jax.numpy as jnp
from jax import lax
from jax.experimental import pallas as pl
from jax.experimental.pallas import tpu as pltpu
from jax.experimental.pallas import tpu_sc as plsc

N_NODES = 10000
N_EDGES = 320000
D = 128

NC = 2
NS = 16
NW = NC * NS

E_PER_W = N_EDGES // NW
CHUNK = 80
NCHUNK = E_PER_W // CHUNK
NSLOT = 4
ROWS_A = 624
TAIL = N_NODES - NS * ROWS_A
ZROWS = 16


def _sc_segment_sum(x, ei):
  mesh = plsc.VectorSubcoreMesh(core_axis_name="c", subcore_axis_name="s")

  @functools.partial(
      pl.kernel,
      mesh=mesh,
      out_type=jax.ShapeDtypeStruct((NC, N_NODES, D), jnp.float32),
      scratch_types=dict(
          sring=pltpu.VMEM((NSLOT * CHUNK,), jnp.int32),
          dring=pltpu.VMEM((NSLOT, CHUNK), jnp.int32),
          rows0=pltpu.VMEM((CHUNK, D), jnp.float32),
          rows1=pltpu.VMEM((CHUNK, D), jnp.float32),
          rows2=pltpu.VMEM((CHUNK, D), jnp.float32),
          rows3=pltpu.VMEM((CHUNK, D), jnp.float32),
          zbuf=pltpu.VMEM((ZROWS, D), jnp.float32),
          acc=pltpu.VMEM_SHARED((N_NODES, D), jnp.float32),
          g0=pltpu.SemaphoreType.DMA,
          g1=pltpu.SemaphoreType.DMA,
          g2=pltpu.SemaphoreType.DMA,
          g3=pltpu.SemaphoreType.DMA,
          s0=pltpu.SemaphoreType.DMA,
          s1=pltpu.SemaphoreType.DMA,
          s2=pltpu.SemaphoreType.DMA,
          s3=pltpu.SemaphoreType.DMA,
          si=pltpu.SemaphoreType.DMA,
          sd=pltpu.SemaphoreType.DMA,
          semz=pltpu.SemaphoreType.DMA,
      ),
  )
  def seg_sum(x_hbm, ei_hbm, out_hbm, *, sring, dring, rows0,
              rows1, rows2, rows3, zbuf, acc, g0, g1, g2, g3, s0, s1, s2,
              s3, si, sd, semz):
    c = lax.axis_index("c")
    s_ax = lax.axis_index("s")
    wid = s_ax * NC + c
    rows = [rows0, rows1, rows2, rows3]
    semg = [g0, g1, g2, g3]
    sems = [s0, s1, s2, s3]

    def sl(k):
      return sring.at[pl.ds(k * CHUNK, CHUNK)]

    def load_pair(jj, m):
      off = pl.multiple_of(wid * E_PER_W + jj * CHUNK, CHUNK)
      pltpu.async_copy(ei_hbm.at[pl.ds(off, CHUNK)], sl(m), si)
      pltpu.async_copy(ei_hbm.at[pl.ds(off + N_EDGES, CHUNK)], dring.at[m],
                       sd)

    def drain_si():
      pltpu.make_async_copy(ei_hbm.at[pl.ds(0, CHUNK)], sl(0), si).wait()

    def drain_sd():
      pltpu.make_async_copy(ei_hbm.at[pl.ds(0, CHUNK)], dring.at[0],
                            sd).wait()

    def g_issue(k):
      pltpu.async_copy(x_hbm.at[sl(k)], rows[k], semg[k])

    def g_wait(k):
      pltpu.make_async_copy(x_hbm.at[sl(k)], rows[k], semg[k]).wait()

    def s_issue(k):
      pltpu.async_copy(rows[k], acc.at[dring.at[k]], sems[k], add=True)

    def s_wait(k):
      pltpu.make_async_copy(rows[k], acc.at[dring.at[k]], sems[k]).wait()

    for t in range(3):
      load_pair(t, t)
    drain_si()
    g_issue(0)
    drain_si()
    g_issue(1)

    for i in range(ZROWS):
      for k in range(D // 16):
        zbuf[i, pl.ds(k * 16, 16)] = jnp.zeros((16,), jnp.float32)
    base = s_ax * ROWS_A
    zcopies = [
        pltpu.async_copy(zbuf, acc.at[pl.ds(base + t * ZROWS, ZROWS)], semz)
        for t in range(ROWS_A // ZROWS)
    ]

    @pl.when(s_ax == NS - 1)
    def _zero_tail():
      pltpu.async_copy(zbuf.at[pl.ds(0, TAIL)],
                       acc.at[pl.ds(NS * ROWS_A, TAIL)], semz).wait()

    for zc in zcopies:
      zc.wait()
    plsc.subcore_barrier()

    g_wait(0)
    drain_sd()
    s_issue(0)
    load_pair(3, 3)
    drain_si()
    g_issue(2)

    def body(i, _):
      jb = 4 * i + 1
      for kk in range(4):
        j = jb + kk
        k = (1 + kk) % 4
        g_wait(k)
        drain_sd()
        s_issue(k)
        s_wait((k - 1) % 4)
        load_pair(j + 3, (k - 1) % 4)
        drain_si()
        g_issue((k + 2) % 4)
      return 0
    lax.fori_loop(0, 30, body, 0)

    g_wait(1)
    drain_sd()
    s_issue(1)
    s_wait(0)
    load_pair(124, 0)
    drain_si()
    g_issue(3)

    g_wait(2)
    drain_sd()
    s_issue(2)
    s_wait(1)
    drain_si()
    g_issue(0)

    g_wait(3)
    drain_sd()
    s_issue(3)
    s_wait(2)

    g_wait(0)
    drain_sd()
    s_issue(0)
    s_wait(3)
    s_wait(0)

    plsc.subcore_barrier()
    pltpu.sync_copy(acc.at[pl.ds(base, ROWS_A)],
                    out_hbm.at[c, pl.ds(base, ROWS_A)])

    @pl.when(s_ax == NS - 1)
    def _copy_tail():
      pltpu.sync_copy(acc.at[pl.ds(NS * ROWS_A, TAIL)],
                      out_hbm.at[c, pl.ds(NS * ROWS_A, TAIL)])

  return seg_sum(x, ei)


def _tc_linear(p, W, b2):
  blk = 1000

  def body(p_ref, w_ref, b_ref, o_ref):
    h = p_ref[0] + p_ref[1]
    o_ref[...] = lax.dot_general(
        h, w_ref[...], (((1,), (1,)), ((), ())),
        preferred_element_type=jnp.float32) + b_ref[...]

  return pl.pallas_call(
      body,
      grid=(N_NODES // blk,),
      in_specs=[
          pl.BlockSpec((NC, blk, D), lambda i: (0, i, 0)),
          pl.BlockSpec((D, D), lambda i: (0, 0)),
          pl.BlockSpec((1, D), lambda i: (0, 0)),
      ],
      out_specs=pl.BlockSpec((blk, D), lambda i: (i, 0)),
      out_shape=jax.ShapeDtypeStruct((N_NODES, D), jnp.float32),
  )(p, W, b2)


@jax.jit
def kernel(inputs, edge_index, W, b):
  partials = _sc_segment_sum(inputs, edge_index.reshape(2 * N_EDGES))
  return _tc_linear(partials, W, b.reshape(1, D))


# speedup vs baseline: 14.6894x; 1.0174x over previous
import functools

import jax
import jax.numpy as jnp
from jax import lax
from jax.experimental import pallas as pl
from jax.experimental.pallas import tpu as pltpu
from jax.experimental.pallas import tpu_sc as plsc

N_NODES = 10000
N_EDGES = 320000
D = 128

NC = 2
NS = 16
NW = NC * NS

E_PER_W = N_EDGES // NW
CHUNK = 80
NCHUNK = E_PER_W // CHUNK
NSLOT = 4
ROWS_A = 624
TAIL = N_NODES - NS * ROWS_A
ZROWS = 48


def _sc_segment_sum(x, ei):
  mesh = plsc.VectorSubcoreMesh(core_axis_name="c", subcore_axis_name="s")

  @functools.partial(
      pl.kernel,
      mesh=mesh,
      out_type=jax.ShapeDtypeStruct((NC, N_NODES, D), jnp.float32),
      scratch_types=dict(
          sring=pltpu.VMEM((NSLOT * CHUNK,), jnp.int32),
          dring=pltpu.VMEM((NSLOT, CHUNK), jnp.int32),
          rows0=pltpu.VMEM((CHUNK, D), jnp.float32),
          rows1=pltpu.VMEM((CHUNK, D), jnp.float32),
          rows2=pltpu.VMEM((CHUNK, D), jnp.float32),
          rows3=pltpu.VMEM((CHUNK, D), jnp.float32),
          zbuf=pltpu.VMEM((ZROWS, D), jnp.float32),
          acc=pltpu.VMEM_SHARED((N_NODES, D), jnp.float32),
          g0=pltpu.SemaphoreType.DMA,
          g1=pltpu.SemaphoreType.DMA,
          g2=pltpu.SemaphoreType.DMA,
          g3=pltpu.SemaphoreType.DMA,
          s0=pltpu.SemaphoreType.DMA,
          s1=pltpu.SemaphoreType.DMA,
          s2=pltpu.SemaphoreType.DMA,
          s3=pltpu.SemaphoreType.DMA,
          si=pltpu.SemaphoreType.DMA,
          sd=pltpu.SemaphoreType.DMA,
          semz=pltpu.SemaphoreType.DMA,
      ),
  )
  def seg_sum(x_hbm, ei_hbm, out_hbm, *, sring, dring, rows0,
              rows1, rows2, rows3, zbuf, acc, g0, g1, g2, g3, s0, s1, s2,
              s3, si, sd, semz):
    c = lax.axis_index("c")
    s_ax = lax.axis_index("s")
    wid = s_ax * NC + c
    rows = [rows0, rows1, rows2, rows3]
    semg = [g0, g1, g2, g3]
    sems = [s0, s1, s2, s3]

    def sl(k):
      return sring.at[pl.ds(k * CHUNK, CHUNK)]

    def load_pair(jj, m):
      off = pl.multiple_of(wid * E_PER_W + jj * CHUNK, CHUNK)
      pltpu.async_copy(ei_hbm.at[pl.ds(off, CHUNK)], sl(m), si)
      pltpu.async_copy(ei_hbm.at[pl.ds(off + N_EDGES, CHUNK)], dring.at[m],
                       sd)

    def drain_si():
      pltpu.make_async_copy(ei_hbm.at[pl.ds(0, CHUNK)], sl(0), si).wait()

    def drain_sd():
      pltpu.make_async_copy(ei_hbm.at[pl.ds(0, CHUNK)], dring.at[0],
                            sd).wait()

    def g_issue(k):
      pltpu.async_copy(x_hbm.at[sl(k)], rows[k], semg[k])

    def g_wait(k):
      pltpu.make_async_copy(x_hbm.at[sl(k)], rows[k], semg[k]).wait()

    def s_issue(k):
      pltpu.async_copy(rows[k], acc.at[dring.at[k]], sems[k], add=True)

    def s_wait(k):
      pltpu.make_async_copy(rows[k], acc.at[dring.at[k]], sems[k]).wait()

    for t in range(3):
      load_pair(t, t)
    drain_si()
    g_issue(0)
    drain_si()
    g_issue(1)

    for i in range(ZROWS):
      for k in range(D // 16):
        zbuf[i, pl.ds(k * 16, 16)] = jnp.zeros((16,), jnp.float32)
    base = s_ax * ROWS_A
    zcopies = [
        pltpu.async_copy(zbuf, acc.at[pl.ds(base + t * ZROWS, ZROWS)], semz)
        for t in range(ROWS_A // ZROWS)
    ]

    @pl.when(s_ax == NS - 1)
    def _zero_tail():
      pltpu.async_copy(zbuf.at[pl.ds(0, TAIL)],
                       acc.at[pl.ds(NS * ROWS_A, TAIL)], semz).wait()

    for zc in zcopies:
      zc.wait()
    plsc.subcore_barrier()

    g_wait(0)
    drain_sd()
    s_issue(0)
    load_pair(3, 3)
    drain_si()
    g_issue(2)

    def body(i, _):
      jb = 4 * i + 1
      for kk in range(4):
        j = jb + kk
        k = (1 + kk) % 4
        g_wait(k)
        drain_sd()
        s_issue(k)
        s_wait((k - 1) % 4)
        load_pair(j + 3, (k - 1) % 4)
        drain_si()
        g_issue((k + 2) % 4)
      return 0
    lax.fori_loop(0, 30, body, 0)

    g_wait(1)
    drain_sd()
    s_issue(1)
    s_wait(0)
    load_pair(124, 0)
    drain_si()
    g_issue(3)

    g_wait(2)
    drain_sd()
    s_issue(2)
    s_wait(1)
    drain_si()
    g_issue(0)

    g_wait(3)
    drain_sd()
    s_issue(3)
    s_wait(2)

    g_wait(0)
    drain_sd()
    s_issue(0)
    s_wait(3)
    s_wait(0)

    plsc.subcore_barrier()
    pltpu.sync_copy(acc.at[pl.ds(base, ROWS_A)],
                    out_hbm.at[c, pl.ds(base, ROWS_A)])

    @pl.when(s_ax == NS - 1)
    def _copy_tail():
      pltpu.sync_copy(acc.at[pl.ds(NS * ROWS_A, TAIL)],
                      out_hbm.at[c, pl.ds(NS * ROWS_A, TAIL)])

  return seg_sum(x, ei)


def _tc_linear(p, W, b2):
  blk = 2000

  def body(p_ref, w_ref, b_ref, o_ref):
    h = p_ref[0] + p_ref[1]
    o_ref[...] = lax.dot_general(
        h, w_ref[...], (((1,), (1,)), ((), ())),
        preferred_element_type=jnp.float32) + b_ref[...]

  return pl.pallas_call(
      body,
      grid=(N_NODES // blk,),
      in_specs=[
          pl.BlockSpec((NC, blk, D), lambda i: (0, i, 0)),
          pl.BlockSpec((D, D), lambda i: (0, 0)),
          pl.BlockSpec((1, D), lambda i: (0, 0)),
      ],
      out_specs=pl.BlockSpec((blk, D), lambda i: (i, 0)),
      out_shape=jax.ShapeDtypeStruct((N_NODES, D), jnp.float32),
  )(p, W, b2)


@jax.jit
def kernel(inputs, edge_index, W, b):
  partials = _sc_segment_sum(inputs, edge_index.reshape(2 * N_EDGES))
  return _tc_linear(partials, W, b.reshape(1, D))
